# Initial kernel scaffold; baseline (speedup 1.0000x reference)
import jax
import jax.numpy as jnp
from jax.experimental import pallas as pl


def kernel(xyz, points, params):
    raise NotImplementedError("write your pallas kernel here")


# speedup vs baseline: 1.7503x; 1.7503x over previous
import functools
import jax
import jax.numpy as jnp
from jax.experimental import pallas as pl

_NPOINT = 512
_RADII = [0.1, 0.2, 0.4]
_NSAMPLES = [16, 32, 64]
_CPAD = 32


def _index_points(points, idx):
    return jax.vmap(lambda p, i: p[i])(points, idx)


def _square_distance(src, dst):
    return (jnp.sum(src ** 2, -1)[:, :, None]
            + jnp.sum(dst ** 2, -1)[:, None, :]
            - 2.0 * jnp.einsum('bsc,bnc->bsn', src, dst))


def _fps(xyz, npoint):
    B, N, C = xyz.shape
    def body(state, i):
        centroids, distance, farthest = state
        centroids = centroids.at[:, i].set(farthest)
        centroid = jnp.take_along_axis(xyz, farthest[:, None, None], axis=1)
        dist = jnp.sum((xyz - centroid) ** 2, -1)
        distance = jnp.minimum(distance, dist)
        farthest = jnp.argmax(distance, -1).astype(jnp.int32)
        return (centroids, distance, farthest), None
    centroids = jnp.zeros((B, npoint), dtype=jnp.int32)
    distance = jnp.full((B, N), 1e10, dtype=jnp.float32)
    farthest = jnp.zeros((B,), dtype=jnp.int32)
    (centroids, _, _), _ = jax.lax.scan(body, (centroids, distance, farthest),
                                        jnp.arange(npoint))
    return centroids


def _ball_query(radius, nsample, xyz, new_xyz):
    B, N, C = xyz.shape
    S = new_xyz.shape[1]
    sqrdists = _square_distance(new_xyz, xyz)
    group_idx = jnp.broadcast_to(jnp.arange(N, dtype=jnp.int32), (B, S, N))
    group_idx = jnp.where(sqrdists > radius ** 2, N, group_idx)
    group_idx = jnp.sort(group_idx, axis=-1)[:, :, :nsample]
    group_first = jnp.broadcast_to(group_idx[:, :, :1], group_idx.shape)
    group_idx = jnp.where(group_idx == N, group_first, group_idx)
    return group_idx


def _mlp_pool_body(K, SB, g_ref, nx_ref, w1_ref, wx_ref, b1_ref,
                   w2_ref, b2_ref, w3_ref, b3_ref, out_ref):
    c1 = w1_ref.shape[1]
    g = g_ref[0]
    x = g.reshape(K * SB, _CPAD)
    a = jnp.dot(x, w1_ref[...], preferred_element_type=jnp.float32)
    nx = nx_ref[0]
    bm = jnp.dot(nx, wx_ref[...], preferred_element_type=jnp.float32)
    bex = jnp.broadcast_to(bm[None], (K, SB, c1)).reshape(K * SB, c1)
    z = jax.nn.relu(a - bex + b1_ref[...])
    z = jax.nn.relu(jnp.dot(z, w2_ref[...], preferred_element_type=jnp.float32)
                    + b2_ref[...])
    z = jax.nn.relu(jnp.dot(z, w3_ref[...], preferred_element_type=jnp.float32)
                    + b3_ref[...])
    c3 = z.shape[1]
    out_ref[0] = jnp.max(z.reshape(K, SB, c3), axis=0)


def _fold_params(branch):
    folded = []
    for (W, b, gamma, beta) in branch:
        scale = gamma / jnp.sqrt(jnp.ones_like(gamma) + 1e-3)
        folded.append((W * scale[None, :], b * scale + beta))
    return folded


def _branch_mlp_pool(gathered, nxT, branch_params, K):
    B = gathered.shape[0]
    S = gathered.shape[2]
    SB = 128
    (w1, b1), (w2, b2), (w3, b3) = _fold_params(branch_params)
    c1, c2, c3 = w1.shape[1], w2.shape[1], w3.shape[1]
    w1p = jnp.zeros((_CPAD, c1), jnp.float32).at[:w1.shape[0]].set(w1)
    wx = jnp.zeros((8, c1), jnp.float32).at[:3].set(w1[16:19])
    grid = (B, S // SB)
    out = pl.pallas_call(
        functools.partial(_mlp_pool_body, K, SB),
        grid=grid,
        in_specs=[
            pl.BlockSpec((1, K, SB, _CPAD), lambda b, s: (b, 0, s, 0)),
            pl.BlockSpec((1, SB, 8), lambda b, s: (b, s, 0)),
            pl.BlockSpec((_CPAD, c1), lambda b, s: (0, 0)),
            pl.BlockSpec((8, c1), lambda b, s: (0, 0)),
            pl.BlockSpec((1, c1), lambda b, s: (0, 0)),
            pl.BlockSpec((c1, c2), lambda b, s: (0, 0)),
            pl.BlockSpec((1, c2), lambda b, s: (0, 0)),
            pl.BlockSpec((c2, c3), lambda b, s: (0, 0)),
            pl.BlockSpec((1, c3), lambda b, s: (0, 0)),
        ],
        out_specs=pl.BlockSpec((1, SB, c3), lambda b, s: (b, s, 0)),
        out_shape=jax.ShapeDtypeStruct((B, S, c3), jnp.float32),
    )(gathered, nxT, w1p, wx, b1[None], w2, b2[None], w3, b3[None])
    return out


def kernel(xyz, points, params):
    xyz_t = jnp.transpose(xyz, (0, 2, 1))
    pts_t = jnp.transpose(points, (0, 2, 1))
    B, N, _ = xyz_t.shape
    S = _NPOINT

    fps_idx = _fps(xyz_t, S)
    new_xyz = _index_points(xyz_t, fps_idx)
    nxT = jnp.zeros((B, S, 8), jnp.float32).at[:, :, :3].set(new_xyz)

    table = jnp.concatenate(
        [pts_t, xyz_t, jnp.zeros((B, N, _CPAD - 19), jnp.float32)], axis=-1)

    outs = []
    for i, radius in enumerate(_RADII):
        K = _NSAMPLES[i]
        gidx = _ball_query(radius, K, xyz_t, new_xyz)
        gidxT = jnp.transpose(gidx, (0, 2, 1))
        gathered = _index_points(table, gidxT)
        outs.append(_branch_mlp_pool(gathered, nxT, params[i], K))

    new_xyz_out = jnp.transpose(new_xyz, (0, 2, 1))
    new_points_concat = jnp.transpose(jnp.concatenate(outs, axis=-1), (0, 2, 1))
    return new_xyz_out, new_points_concat


# speedup vs baseline: 2.2922x; 1.3096x over previous
import functools
import jax
import jax.numpy as jnp
from jax.experimental import pallas as pl

_NPOINT = 512
_RADII = [0.1, 0.2, 0.4]
_NSAMPLES = [16, 32, 64]
_CPAD = 32


def _index_points(points, idx):
    return jax.vmap(lambda p, i: p[i])(points, idx)


def _square_distance(src, dst):
    return (jnp.sum(src ** 2, -1)[:, :, None]
            + jnp.sum(dst ** 2, -1)[:, None, :]
            - 2.0 * jnp.einsum('bsc,bnc->bsn', src, dst))


def _fps_body(B, N, S, x_ref, idx_ref):
    x = x_ref[...]
    iota = jax.lax.broadcasted_iota(jnp.int32, (1, N), 1)

    def body(i, carry):
        dists, fars = carry
        ndists = []
        nfars = []
        for b in range(B):
            idx_ref[b, i] = fars[b]
            xb = x[8 * b:8 * b + 8]
            c = jnp.sum(jnp.where(iota == fars[b], xb, 0.0), axis=1,
                        keepdims=True)
            d = jnp.sum((xb - c) ** 2, axis=0, keepdims=True)
            nd = jnp.minimum(dists[b], d)
            m = jnp.max(nd)
            nf = jnp.min(jnp.where(nd == m, iota, N))
            ndists.append(nd)
            nfars.append(nf)
        return (tuple(ndists), tuple(nfars))

    dists0 = tuple(jnp.full((1, N), 1e10, jnp.float32) for _ in range(B))
    fars0 = tuple(jnp.int32(0) for _ in range(B))
    jax.lax.fori_loop(0, S, body, (dists0, fars0))


def _fps_idx(xyz, npoint):
    from jax.experimental.pallas import tpu as pltpu
    B, _, N = xyz.shape
    xpad = jnp.zeros((B, 8, N), jnp.float32).at[:, :3].set(xyz)
    return pl.pallas_call(
        functools.partial(_fps_body, B, N, npoint),
        out_specs=pl.BlockSpec(memory_space=pltpu.MemorySpace.SMEM),
        out_shape=jax.ShapeDtypeStruct((B, npoint), jnp.int32),
    )(xpad.reshape(8 * B, N))


def _ball_query(radius, nsample, xyz, new_xyz):
    B, N, C = xyz.shape
    S = new_xyz.shape[1]
    sqrdists = _square_distance(new_xyz, xyz)
    group_idx = jnp.broadcast_to(jnp.arange(N, dtype=jnp.int32), (B, S, N))
    group_idx = jnp.where(sqrdists > radius ** 2, N, group_idx)
    group_idx = jnp.sort(group_idx, axis=-1)[:, :, :nsample]
    group_first = jnp.broadcast_to(group_idx[:, :, :1], group_idx.shape)
    group_idx = jnp.where(group_idx == N, group_first, group_idx)
    return group_idx


def _mlp_pool_body(K, SB, g_ref, nx_ref, w1_ref, wx_ref, b1_ref,
                   w2_ref, b2_ref, w3_ref, b3_ref, out_ref):
    c1 = w1_ref.shape[1]
    g = g_ref[0]
    x = g.reshape(K * SB, _CPAD)
    a = jnp.dot(x, w1_ref[...], preferred_element_type=jnp.float32)
    nx = nx_ref[0]
    bm = jnp.dot(nx, wx_ref[...], preferred_element_type=jnp.float32)
    bex = jnp.broadcast_to(bm[None], (K, SB, c1)).reshape(K * SB, c1)
    z = jax.nn.relu(a - bex + b1_ref[...])
    z = jax.nn.relu(jnp.dot(z, w2_ref[...], preferred_element_type=jnp.float32)
                    + b2_ref[...])
    z = jax.nn.relu(jnp.dot(z, w3_ref[...], preferred_element_type=jnp.float32)
                    + b3_ref[...])
    c3 = z.shape[1]
    out_ref[0] = jnp.max(z.reshape(K, SB, c3), axis=0)


def _fold_params(branch):
    folded = []
    for (W, b, gamma, beta) in branch:
        scale = gamma / jnp.sqrt(jnp.ones_like(gamma) + 1e-3)
        folded.append((W * scale[None, :], b * scale + beta))
    return folded


def _branch_mlp_pool(gathered, nxT, branch_params, K):
    B = gathered.shape[0]
    S = gathered.shape[2]
    SB = 128
    (w1, b1), (w2, b2), (w3, b3) = _fold_params(branch_params)
    c1, c2, c3 = w1.shape[1], w2.shape[1], w3.shape[1]
    w1p = jnp.zeros((_CPAD, c1), jnp.float32).at[:w1.shape[0]].set(w1)
    wx = jnp.zeros((8, c1), jnp.float32).at[:3].set(w1[16:19])
    grid = (B, S // SB)
    out = pl.pallas_call(
        functools.partial(_mlp_pool_body, K, SB),
        grid=grid,
        in_specs=[
            pl.BlockSpec((1, K, SB, _CPAD), lambda b, s: (b, 0, s, 0)),
            pl.BlockSpec((1, SB, 8), lambda b, s: (b, s, 0)),
            pl.BlockSpec((_CPAD, c1), lambda b, s: (0, 0)),
            pl.BlockSpec((8, c1), lambda b, s: (0, 0)),
            pl.BlockSpec((1, c1), lambda b, s: (0, 0)),
            pl.BlockSpec((c1, c2), lambda b, s: (0, 0)),
            pl.BlockSpec((1, c2), lambda b, s: (0, 0)),
            pl.BlockSpec((c2, c3), lambda b, s: (0, 0)),
            pl.BlockSpec((1, c3), lambda b, s: (0, 0)),
        ],
        out_specs=pl.BlockSpec((1, SB, c3), lambda b, s: (b, s, 0)),
        out_shape=jax.ShapeDtypeStruct((B, S, c3), jnp.float32),
    )(gathered, nxT, w1p, wx, b1[None], w2, b2[None], w3, b3[None])
    return out


def kernel(xyz, points, params):
    xyz_t = jnp.transpose(xyz, (0, 2, 1))
    pts_t = jnp.transpose(points, (0, 2, 1))
    B, N, _ = xyz_t.shape
    S = _NPOINT

    fps_idx = _fps_idx(xyz, S)
    new_xyz = _index_points(xyz_t, fps_idx)
    nxT = jnp.zeros((B, S, 8), jnp.float32).at[:, :, :3].set(new_xyz)

    table = jnp.concatenate(
        [pts_t, xyz_t, jnp.zeros((B, N, _CPAD - 19), jnp.float32)], axis=-1)

    outs = []
    for i, radius in enumerate(_RADII):
        K = _NSAMPLES[i]
        gidx = _ball_query(radius, K, xyz_t, new_xyz)
        gidxT = jnp.transpose(gidx, (0, 2, 1))
        gathered = _index_points(table, gidxT)
        outs.append(_branch_mlp_pool(gathered, nxT, params[i], K))

    new_xyz_out = jnp.transpose(new_xyz, (0, 2, 1))
    new_points_concat = jnp.transpose(jnp.concatenate(outs, axis=-1), (0, 2, 1))
    return new_xyz_out, new_points_concat
